# Initial kernel scaffold; baseline (speedup 1.0000x reference)
import functools

import jax
import jax.numpy as jnp
from jax import lax
from jax.experimental import pallas as pl
from jax.experimental.pallas import tpu as pltpu
from jax.experimental.pallas import tpu_sc as plsc

N = 10000
D = 128
E = 320000
EP = E + N
NC = 2
NS = 16
LN = 16
NV = 1300
SA = NV * LN
E_PAD = SA * NS
NVH = NV // 2
DEN_ROWS = 640
NROW_BLK = 625
CHUNK = 65

_mesh = plsc.VectorSubcoreMesh(core_axis_name="c", subcore_axis_name="s")


def _make_sc_layer():
    @functools.partial(
        pl.kernel,
        out_type=(
            jax.ShapeDtypeStruct((NC, N, D), jnp.float32),
            jax.ShapeDtypeStruct((NS * NV, LN), jnp.float32),
        ),
        mesh=_mesh,
        scratch_types=[
            pltpu.VMEM((N,), jnp.float32),
            pltpu.VMEM((N,), jnp.float32),
            pltpu.VMEM((DEN_ROWS, LN), jnp.float32),
            pltpu.VMEM((NV, LN), jnp.int32),
            pltpu.VMEM((NV, LN), jnp.int32),
            pltpu.VMEM((NV, LN), jnp.float32),
            pltpu.VMEM((LN, D), jnp.float32),
            pltpu.VMEM((25, D), jnp.float32),
            pltpu.VMEM((LN,), jnp.float32),
            pltpu.VMEM((CHUNK, LN), jnp.float32),
            pltpu.VMEM((5, 128), jnp.int32),
            pltpu.VMEM_SHARED((DEN_ROWS, LN), jnp.float32),
            pltpu.VMEM_SHARED((N, D), jnp.float32),
        ],
    )
    def sc_layer(h_hbm, as_hbm, ad_hbm, src_hbm, dst_hbm,
                 acc_out, attn_out,
                 as_t, ad_t, den_l, src_l, dst_l, ex_l,
                 rows, zbuf, abuf, attn_buf, iota2d, den_sh, acc_sh):
        cid = lax.axis_index("c")
        sid = lax.axis_index("s")
        zero16 = jnp.zeros((LN,), jnp.float32)
        i16 = jnp.arange(LN, dtype=jnp.int32)

        pltpu.sync_copy(as_hbm, as_t)
        pltpu.sync_copy(ad_hbm, ad_t)
        pltpu.sync_copy(src_hbm.at[pl.ds(sid * NV, NV), :], src_l)
        pltpu.sync_copy(dst_hbm.at[pl.ds(sid * NV, NV), :], dst_l)

        @pl.loop(0, DEN_ROWS)
        def _(i):
            den_l[i, :] = zero16

        @pl.loop(0, 25)
        def _(i):
            for c in range(8):
                zbuf[i, pl.ds(c * LN, LN)] = zero16

        for j in range(5):
            for k in range(8):
                iota2d[j, pl.ds(k * LN, LN)] = i16 + (j * 128 + k * LN)

        @pl.when(sid == 0)
        def _():
            pltpu.sync_copy(den_l, den_sh)

        @pl.loop(0, 25)
        def _(t):
            pltpu.sync_copy(
                zbuf, acc_sh.at[pl.ds(sid * NROW_BLK + t * 25, 25), :])

        plsc.subcore_barrier()

        def _mx(i, m):
            return jnp.maximum(m, as_t[pl.ds(i * LN, LN)])

        def _mx2(i, m):
            return jnp.maximum(m, ad_t[pl.ds(i * LN, LN)])

        neg = jnp.full((LN,), -1e30, jnp.float32)
        ms = jnp.max(lax.fori_loop(0, N // LN, _mx, neg))
        md = jnp.max(lax.fori_loop(0, N // LN, _mx2, neg))
        shift = jnp.maximum(ms + md, 0.0)

        base_e = sid * SA

        @pl.loop(0, NV)
        def _(v):
            s = src_l[v, :]
            d = dst_l[v, :]
            a = plsc.load_gather(as_t, [s]) + plsc.load_gather(ad_t, [d])
            a = jnp.where(a >= 0.0, a, 0.2 * a)
            ex = jnp.exp(a - shift)
            ge = base_e + v * LN + i16
            ex = jnp.where(ge < EP, ex, 0.0)
            ex_l[v, :] = ex
            plsc.addupdate_scatter(
                den_l, [lax.shift_right_logical(d, 4),
                        jnp.bitwise_and(d, 15)], ex)

        for j in range(5):
            pltpu.sync_copy(den_l.at[pl.ds(j * 128, 128), :],
                            den_sh.at[iota2d.at[j]], add=True)

        plsc.subcore_barrier()

        pltpu.sync_copy(den_sh, den_l)

        @pl.loop(0, DEN_ROWS)
        def _(i):
            den_l[i, :] = 1.0 / (den_l[i, :] + 1e-16)

        vbase = cid * NVH
        abase = sid * NV + vbase

        @pl.loop(0, NVH // CHUNK)
        def _(k):
            @pl.loop(0, CHUNK)
            def _(w):
                v = vbase + k * CHUNK + w
                d = dst_l[v, :]
                inv = plsc.load_gather(
                    den_l, [lax.shift_right_logical(d, 4),
                            jnp.bitwise_and(d, 15)])
                attn = ex_l[v, :] * inv
                attn_buf[w, :] = attn
                abuf[:] = attn
                pltpu.sync_copy(h_hbm.at[src_l.at[v]], rows)
                for i in range(LN):
                    ai = plsc.load_gather(
                        abuf, [jnp.full((LN,), i, jnp.int32)])
                    for c in range(8):
                        sl = pl.ds(c * LN, LN)
                        rows[i, sl] = rows[i, sl] * ai
                pltpu.sync_copy(rows, acc_sh.at[dst_l.at[v]], add=True)

            pltpu.sync_copy(attn_buf,
                            attn_out.at[pl.ds(abase + k * CHUNK, CHUNK), :])

        plsc.subcore_barrier()

        pltpu.sync_copy(acc_sh.at[pl.ds(sid * NROW_BLK, NROW_BLK), :],
                        acc_out.at[cid, pl.ds(sid * NROW_BLK, NROW_BLK), :])

    return sc_layer


_sc_layer = _make_sc_layer()

_BLK = 1000


def _tc1_body(x_ref, w_ref, a_ref, h_ref, pair_ref):
    h = jnp.dot(x_ref[...], w_ref[...], preferred_element_type=jnp.float32)
    h_ref[...] = h
    pair_ref[...] = jnp.dot(h, a_ref[...], preferred_element_type=jnp.float32)


def _tc1(x, w, a):
    return pl.pallas_call(
        _tc1_body,
        grid=(N // _BLK,),
        in_specs=[
            pl.BlockSpec((_BLK, D), lambda i: (i, 0)),
            pl.BlockSpec((D, D), lambda i: (0, 0)),
            pl.BlockSpec((D, D), lambda i: (0, 0)),
        ],
        out_specs=[
            pl.BlockSpec((_BLK, D), lambda i: (i, 0)),
            pl.BlockSpec((_BLK, D), lambda i: (i, 0)),
        ],
        out_shape=[
            jax.ShapeDtypeStruct((N, D), jnp.float32),
            jax.ShapeDtypeStruct((N, D), jnp.float32),
        ],
    )(x, w, a)


def _tc2_body(p0_ref, p1_ref, b_ref, w_ref, a_ref, h_ref, pair_ref):
    hin = jax.nn.relu(p0_ref[...] + p1_ref[...] + b_ref[...])
    h = jnp.dot(hin, w_ref[...], preferred_element_type=jnp.float32)
    h_ref[...] = h
    pair_ref[...] = jnp.dot(h, a_ref[...], preferred_element_type=jnp.float32)


def _tc2(p0, p1, b, w, a):
    return pl.pallas_call(
        _tc2_body,
        grid=(N // _BLK,),
        in_specs=[
            pl.BlockSpec((_BLK, D), lambda i: (i, 0)),
            pl.BlockSpec((_BLK, D), lambda i: (i, 0)),
            pl.BlockSpec((1, D), lambda i: (0, 0)),
            pl.BlockSpec((D, D), lambda i: (0, 0)),
            pl.BlockSpec((D, D), lambda i: (0, 0)),
        ],
        out_specs=[
            pl.BlockSpec((_BLK, D), lambda i: (i, 0)),
            pl.BlockSpec((_BLK, D), lambda i: (i, 0)),
        ],
        out_shape=[
            jax.ShapeDtypeStruct((N, D), jnp.float32),
            jax.ShapeDtypeStruct((N, D), jnp.float32),
        ],
    )(p0, p1, b, w, a)


def _tc3_body(p0_ref, p1_ref, b_ref, o_ref):
    o_ref[...] = jax.nn.relu(p0_ref[...] + p1_ref[...] + b_ref[...])


def _tc3(p0, p1, b):
    return pl.pallas_call(
        _tc3_body,
        grid=(N // _BLK,),
        in_specs=[
            pl.BlockSpec((_BLK, D), lambda i: (i, 0)),
            pl.BlockSpec((_BLK, D), lambda i: (i, 0)),
            pl.BlockSpec((1, D), lambda i: (0, 0)),
        ],
        out_specs=pl.BlockSpec((_BLK, D), lambda i: (i, 0)),
        out_shape=jax.ShapeDtypeStruct((N, D), jnp.float32),
    )(p0, p1, b)


def _proj_mat(a_src, a_dst):
    m = jnp.zeros((D, D), jnp.float32)
    m = m.at[:, 0].set(a_src)
    m = m.at[:, 1].set(a_dst)
    return m


@jax.jit
def kernel(x, edge_index, W1, a_src1, a_dst1, b1, W2, a_src2, a_dst2, b2):
    loop = jnp.arange(N, dtype=jnp.int32)
    pad = jnp.zeros((E_PAD - EP,), jnp.int32)
    src2d = jnp.concatenate([edge_index[0], loop, pad]).reshape(NS * NV, LN)
    dst2d = jnp.concatenate([edge_index[1], loop, pad]).reshape(NS * NV, LN)

    h1, pair1 = _tc1(x, W1, _proj_mat(a_src1, a_dst1))
    acc1, _ = _sc_layer(h1, pair1[:, 0], pair1[:, 1], src2d, dst2d)
    h2, pair2 = _tc2(acc1[0], acc1[1], b1.reshape(1, D), W2,
                     _proj_mat(a_src2, a_dst2))
    acc2, attn2 = _sc_layer(h2, pair2[:, 0], pair2[:, 1], src2d, dst2d)
    out = _tc3(acc2[0], acc2[1], b2.reshape(1, D))
    attn = attn2.reshape(-1)[:EP][:, None]
    return out, attn


# speedup vs baseline: 14.2583x; 14.2583x over previous
import functools

import jax
import jax.numpy as jnp
from jax import lax
from jax.experimental import pallas as pl
from jax.experimental.pallas import tpu as pltpu
from jax.experimental.pallas import tpu_sc as plsc

N = 10000
D = 128
E = 320000
EP = E + N
NC = 2
NS = 16
LN = 16
NV = 1312
SA = NV * LN
E_PAD = SA * NS
DEN_ROWS = 640
NHALF = N // 2
ACC_N = 5120
NROW_BLK = ACC_N // NS
CHA = 164
NCHA = NV // CHA
CCAP = 12288

_mesh = plsc.VectorSubcoreMesh(core_axis_name="c", subcore_axis_name="s")

_sc_params = pltpu.CompilerParams(
    needs_layout_passes=False, use_tc_tiling_on_sc=False)


def _make_sc_layer():
    @functools.partial(
        pl.kernel,
        out_type=(
            jax.ShapeDtypeStruct((NC, ACC_N, D), jnp.float32),
            jax.ShapeDtypeStruct((NS * NV, LN), jnp.float32),
        ),
        mesh=_mesh,
        compiler_params=_sc_params,
        scratch_types=[
            pltpu.VMEM((N,), jnp.float32),
            pltpu.VMEM((N,), jnp.float32),
            pltpu.VMEM((DEN_ROWS, LN), jnp.float32),
            pltpu.VMEM((CHA, LN), jnp.int32),
            pltpu.VMEM((CHA, LN), jnp.int32),
            pltpu.VMEM((CHA, LN), jnp.float32),
            pltpu.VMEM((CCAP + LN,), jnp.int32),
            pltpu.VMEM((CCAP + LN,), jnp.int32),
            pltpu.VMEM((CCAP + LN,), jnp.float32),
            pltpu.VMEM((LN, D), jnp.float32),
            pltpu.VMEM((5, 128), jnp.int32),
            pltpu.VMEM_SHARED((DEN_ROWS, LN), jnp.float32),
            pltpu.VMEM_SHARED((ACC_N, D), jnp.float32),
        ],
    )
    def sc_layer(h_hbm, as_hbm, ad_hbm, src_hbm, dst_hbm,
                 acc_out, attn_out,
                 as_t, ad_t, den_l, src_ch, dst_ch, attn_ch,
                 src_c, dstl_c, attn_c, rows, iota2d,
                 den_sh, acc_sh):
        cid = lax.axis_index("c")
        sid = lax.axis_index("s")
        zero16f = jnp.zeros((LN,), jnp.float32)
        zero16i = jnp.zeros((LN,), jnp.int32)
        i16 = jnp.arange(LN, dtype=jnp.int32)

        pltpu.sync_copy(as_hbm, as_t)
        pltpu.sync_copy(ad_hbm, ad_t)

        @pl.loop(0, DEN_ROWS)
        def _(i):
            den_l[i, :] = zero16f

        @pl.loop(0, LN)
        def _(i):
            for c in range(8):
                rows[i, pl.ds(c * LN, LN)] = zero16f

        for j in range(5):
            for k in range(8):
                iota2d[j, pl.ds(k * LN, LN)] = i16 + (j * 128 + k * LN)

        @pl.when(sid == 0)
        def _():
            pltpu.sync_copy(den_l, den_sh)

        @pl.loop(0, NROW_BLK // LN)
        def _(t):
            pltpu.sync_copy(
                rows, acc_sh.at[pl.ds(sid * NROW_BLK + t * LN, LN), :])

        plsc.subcore_barrier()

        def _mx(i, m):
            return jnp.maximum(m, as_t[pl.ds(i * LN, LN)])

        def _mx2(i, m):
            return jnp.maximum(m, ad_t[pl.ds(i * LN, LN)])

        neg = jnp.full((LN,), -1e30, jnp.float32)
        ms = jnp.max(lax.fori_loop(0, N // LN, _mx, neg))
        md = jnp.max(lax.fori_loop(0, N // LN, _mx2, neg))
        shift = jnp.maximum(ms + md, 0.0)

        @pl.loop(0, NCHA)
        def _(t):
            pltpu.sync_copy(src_hbm.at[pl.ds(sid * NV + t * CHA, CHA), :],
                            src_ch)
            pltpu.sync_copy(dst_hbm.at[pl.ds(sid * NV + t * CHA, CHA), :],
                            dst_ch)

            @pl.loop(0, CHA)
            def _(w):
                v = t * CHA + w
                s = src_ch[w, :]
                d = dst_ch[w, :]
                a = plsc.load_gather(as_t, [s]) + plsc.load_gather(ad_t, [d])
                a = jnp.where(a >= 0.0, a, 0.2 * a)
                ex = jnp.exp(a - shift)
                ge = (sid * NV + v) * LN + i16
                ex = jnp.where(ge < EP, ex, 0.0)
                plsc.addupdate_scatter(
                    den_l, [lax.shift_right_logical(d, 4),
                            jnp.bitwise_and(d, 15)], ex)

        for j in range(5):
            pltpu.sync_copy(den_l.at[pl.ds(j * 128, 128), :],
                            den_sh.at[iota2d.at[j]], add=True)

        plsc.subcore_barrier()

        pltpu.sync_copy(den_sh, den_l)

        @pl.loop(0, DEN_ROWS)
        def _(i):
            den_l[i, :] = 1.0 / (den_l[i, :] + 1e-16)

        lo = cid * NHALF

        def _chunk(t, off):
            pltpu.sync_copy(src_hbm.at[pl.ds(sid * NV + t * CHA, CHA), :],
                            src_ch)
            pltpu.sync_copy(dst_hbm.at[pl.ds(sid * NV + t * CHA, CHA), :],
                            dst_ch)

            def _vec(w, off):
                v = t * CHA + w
                s = src_ch[w, :]
                d = dst_ch[w, :]
                a = plsc.load_gather(as_t, [s]) + plsc.load_gather(ad_t, [d])
                a = jnp.where(a >= 0.0, a, 0.2 * a)
                ex = jnp.exp(a - shift)
                ge = (sid * NV + v) * LN + i16
                ex = jnp.where(ge < EP, ex, 0.0)
                inv = plsc.load_gather(
                    den_l, [lax.shift_right_logical(d, 4),
                            jnp.bitwise_and(d, 15)])
                attn = ex * inv
                attn_ch[w, :] = attn
                m = (d >= lo) & (d < lo + NHALF) & (ge < EP)
                cum = plsc.cumsum(m.astype(jnp.int32))
                pos = off + cum - 1
                plsc.store_scatter(src_c, [pos], s, mask=m)
                plsc.store_scatter(dstl_c, [pos], d - lo, mask=m)
                plsc.store_scatter(attn_c, [pos], attn, mask=m)
                return jnp.minimum(off + jnp.max(cum), CCAP)

            off = lax.fori_loop(0, CHA, _vec, off)

            @pl.when(cid == 0)
            def _():
                pltpu.sync_copy(
                    attn_ch,
                    attn_out.at[pl.ds(sid * NV + t * CHA, CHA), :])

            return off

        off = lax.fori_loop(0, NCHA, _chunk, jnp.int32(0))

        src_c[pl.ds(off, LN)] = zero16i
        dstl_c[pl.ds(off, LN)] = zero16i
        attn_c[pl.ds(off, LN)] = zero16f
        ngrp = lax.shift_right_logical(off + LN - 1, 4)

        @pl.loop(0, ngrp)
        def _(g):
            base = g * LN
            s = src_c[pl.ds(base, LN)]
            dl = dstl_c[pl.ds(base, LN)]
            attn = attn_c[pl.ds(base, LN)]
            pltpu.sync_copy(h_hbm.at[s], rows)
            for i in range(LN):
                ai = lax.gather(
                    attn, jnp.full((LN, 1), i, jnp.int32),
                    lax.GatherDimensionNumbers(
                        offset_dims=(), collapsed_slice_dims=(0,),
                        start_index_map=(0,)),
                    slice_sizes=(1,),
                    mode=lax.GatherScatterMode.PROMISE_IN_BOUNDS)
                for c in range(8):
                    sl = pl.ds(c * LN, LN)
                    rows[i, sl] = rows[i, sl] * ai
            pltpu.sync_copy(rows, acc_sh.at[dl], add=True)

        plsc.subcore_barrier()

        pltpu.sync_copy(acc_sh.at[pl.ds(sid * NROW_BLK, NROW_BLK), :],
                        acc_out.at[cid, pl.ds(sid * NROW_BLK, NROW_BLK), :])

    return sc_layer


_sc_layer = _make_sc_layer()

_BLK = 1000


def _acc_spec():
    return pl.BlockSpec((1, _BLK, D), lambda i: (i // 5, i % 5, 0))


def _tc1_body(x_ref, w_ref, a_ref, h_ref, pair_ref):
    h = jnp.dot(x_ref[...], w_ref[...], preferred_element_type=jnp.float32)
    h_ref[...] = h
    pair_ref[...] = jnp.dot(h, a_ref[...], preferred_element_type=jnp.float32)


def _tc1(x, w, a):
    return pl.pallas_call(
        _tc1_body,
        grid=(N // _BLK,),
        in_specs=[
            pl.BlockSpec((_BLK, D), lambda i: (i, 0)),
            pl.BlockSpec((D, D), lambda i: (0, 0)),
            pl.BlockSpec((D, D), lambda i: (0, 0)),
        ],
        out_specs=[
            pl.BlockSpec((_BLK, D), lambda i: (i, 0)),
            pl.BlockSpec((_BLK, D), lambda i: (i, 0)),
        ],
        out_shape=[
            jax.ShapeDtypeStruct((N, D), jnp.float32),
            jax.ShapeDtypeStruct((N, D), jnp.float32),
        ],
    )(x, w, a)


def _tc2_body(acc_ref, b_ref, w_ref, a_ref, h_ref, pair_ref):
    hin = jax.nn.relu(acc_ref[0] + b_ref[...])
    h = jnp.dot(hin, w_ref[...], preferred_element_type=jnp.float32)
    h_ref[...] = h
    pair_ref[...] = jnp.dot(h, a_ref[...], preferred_element_type=jnp.float32)


def _tc2(acc, b, w, a):
    return pl.pallas_call(
        _tc2_body,
        grid=(N // _BLK,),
        in_specs=[
            _acc_spec(),
            pl.BlockSpec((1, D), lambda i: (0, 0)),
            pl.BlockSpec((D, D), lambda i: (0, 0)),
            pl.BlockSpec((D, D), lambda i: (0, 0)),
        ],
        out_specs=[
            pl.BlockSpec((_BLK, D), lambda i: (i, 0)),
            pl.BlockSpec((_BLK, D), lambda i: (i, 0)),
        ],
        out_shape=[
            jax.ShapeDtypeStruct((N, D), jnp.float32),
            jax.ShapeDtypeStruct((N, D), jnp.float32),
        ],
    )(acc, b, w, a)


def _tc3_body(acc_ref, b_ref, o_ref):
    o_ref[...] = jax.nn.relu(acc_ref[0] + b_ref[...])


def _tc3(acc, b):
    return pl.pallas_call(
        _tc3_body,
        grid=(N // _BLK,),
        in_specs=[
            _acc_spec(),
            pl.BlockSpec((1, D), lambda i: (0, 0)),
        ],
        out_specs=pl.BlockSpec((_BLK, D), lambda i: (i, 0)),
        out_shape=jax.ShapeDtypeStruct((N, D), jnp.float32),
    )(acc, b)


def _proj_mat(a_src, a_dst):
    m = jnp.zeros((D, D), jnp.float32)
    m = m.at[:, 0].set(a_src)
    m = m.at[:, 1].set(a_dst)
    return m


@jax.jit
def kernel(x, edge_index, W1, a_src1, a_dst1, b1, W2, a_src2, a_dst2, b2):
    loop = jnp.arange(N, dtype=jnp.int32)
    pad = jnp.zeros((E_PAD - EP,), jnp.int32)
    src2d = jnp.concatenate([edge_index[0], loop, pad]).reshape(NS * NV, LN)
    dst2d = jnp.concatenate([edge_index[1], loop, pad]).reshape(NS * NV, LN)

    h1, pair1 = _tc1(x, W1, _proj_mat(a_src1, a_dst1))
    acc1, _ = _sc_layer(h1, pair1[:, 0], pair1[:, 1], src2d, dst2d)
    h2, pair2 = _tc2(acc1, b1.reshape(1, D), W2, _proj_mat(a_src2, a_dst2))
    acc2, attn2 = _sc_layer(h2, pair2[:, 0], pair2[:, 1], src2d, dst2d)
    out = _tc3(acc2, b2.reshape(1, D))
    attn = attn2.reshape(-1)[:EP][:, None]
    return out, attn


# speedup vs baseline: 27.4519x; 1.9253x over previous
import functools

import jax
import jax.numpy as jnp
from jax import lax
from jax.experimental import pallas as pl
from jax.experimental.pallas import tpu as pltpu
from jax.experimental.pallas import tpu_sc as plsc

N = 10000
D = 128
E = 320000
EP = E + N
NC = 2
NS = 16
LN = 16
NV = 1312
SA = NV * LN
E_PAD = SA * NS
DEN_ROWS = 640
NHALF = N // 2
ACC_N = 5120
NROW_BLK = ACC_N // NS
CHA = 164
NCHA = NV // CHA
CCAP = 12288
CG = 32
GV = CG // LN

_mesh = plsc.VectorSubcoreMesh(core_axis_name="c", subcore_axis_name="s")

_sc_params = pltpu.CompilerParams(
    needs_layout_passes=False, use_tc_tiling_on_sc=False)


def _make_sc_layer():
    @functools.partial(
        pl.kernel,
        out_type=(
            jax.ShapeDtypeStruct((NC, ACC_N, D), jnp.float32),
            jax.ShapeDtypeStruct((NS * NV, LN), jnp.float32),
        ),
        mesh=_mesh,
        compiler_params=_sc_params,
        scratch_types=[
            pltpu.VMEM((N,), jnp.float32),
            pltpu.VMEM((N,), jnp.float32),
            pltpu.VMEM((DEN_ROWS, LN), jnp.float32),
            pltpu.VMEM((CHA, LN), jnp.int32),
            pltpu.VMEM((CHA, LN), jnp.int32),
            pltpu.VMEM((CHA, LN), jnp.float32),
            pltpu.VMEM((CCAP + CG,), jnp.int32),
            pltpu.VMEM((CCAP + CG,), jnp.int32),
            pltpu.VMEM((CCAP + CG,), jnp.float32),
            pltpu.VMEM((CG, D), jnp.float32),
            pltpu.VMEM((CG, D), jnp.float32),
            pltpu.VMEM((5, 128), jnp.int32),
            pltpu.SemaphoreType.DMA,
            pltpu.SemaphoreType.DMA,
            pltpu.VMEM_SHARED((DEN_ROWS, LN), jnp.float32),
            pltpu.VMEM_SHARED((ACC_N, D), jnp.float32),
        ],
    )
    def sc_layer(h_hbm, as_hbm, ad_hbm, src_hbm, dst_hbm,
                 acc_out, attn_out,
                 as_t, ad_t, den_l, src_ch, dst_ch, attn_ch,
                 src_c, dstl_c, attn_c, rows0, rows1, iota2d,
                 sem0, sem1, den_sh, acc_sh):
        cid = lax.axis_index("c")
        sid = lax.axis_index("s")
        zero16f = jnp.zeros((LN,), jnp.float32)
        zero16i = jnp.zeros((LN,), jnp.int32)
        i16 = jnp.arange(LN, dtype=jnp.int32)

        pltpu.sync_copy(as_hbm, as_t)
        pltpu.sync_copy(ad_hbm, ad_t)

        @pl.loop(0, DEN_ROWS)
        def _(i):
            den_l[i, :] = zero16f

        @pl.loop(0, CG)
        def _(i):
            for c in range(8):
                rows0[i, pl.ds(c * LN, LN)] = zero16f

        for j in range(5):
            for k in range(8):
                iota2d[j, pl.ds(k * LN, LN)] = i16 + (j * 128 + k * LN)

        @pl.when(sid == 0)
        def _():
            pltpu.sync_copy(den_l, den_sh)

        @pl.loop(0, NROW_BLK // CG)
        def _(t):
            pltpu.sync_copy(
                rows0, acc_sh.at[pl.ds(sid * NROW_BLK + t * CG, CG), :])

        plsc.subcore_barrier()

        def _mx(i, m):
            return jnp.maximum(m, as_t[pl.ds(i * LN, LN)])

        def _mx2(i, m):
            return jnp.maximum(m, ad_t[pl.ds(i * LN, LN)])

        neg = jnp.full((LN,), -1e30, jnp.float32)
        ms = jnp.max(lax.fori_loop(0, N // LN, _mx, neg))
        md = jnp.max(lax.fori_loop(0, N // LN, _mx2, neg))
        shift = jnp.maximum(ms + md, 0.0)

        @pl.loop(0, NCHA)
        def _(t):
            pltpu.sync_copy(src_hbm.at[pl.ds(sid * NV + t * CHA, CHA), :],
                            src_ch)
            pltpu.sync_copy(dst_hbm.at[pl.ds(sid * NV + t * CHA, CHA), :],
                            dst_ch)

            @pl.loop(0, CHA)
            def _(w):
                v = t * CHA + w
                s = src_ch[w, :]
                d = dst_ch[w, :]
                a = plsc.load_gather(as_t, [s]) + plsc.load_gather(ad_t, [d])
                a = jnp.where(a >= 0.0, a, 0.2 * a)
                ex = jnp.exp(a - shift)
                ge = (sid * NV + v) * LN + i16
                ex = jnp.where(ge < EP, ex, 0.0)
                plsc.addupdate_scatter(
                    den_l, [lax.shift_right_logical(d, 4),
                            jnp.bitwise_and(d, 15)], ex)

        for j in range(5):
            pltpu.sync_copy(den_l.at[pl.ds(j * 128, 128), :],
                            den_sh.at[iota2d.at[j]], add=True)

        plsc.subcore_barrier()

        pltpu.sync_copy(den_sh, den_l)

        @pl.loop(0, DEN_ROWS)
        def _(i):
            den_l[i, :] = 1.0 / (den_l[i, :] + 1e-16)

        lo = cid * NHALF

        def _chunk(t, off):
            pltpu.sync_copy(src_hbm.at[pl.ds(sid * NV + t * CHA, CHA), :],
                            src_ch)
            pltpu.sync_copy(dst_hbm.at[pl.ds(sid * NV + t * CHA, CHA), :],
                            dst_ch)

            def _vec(w, off):
                v = t * CHA + w
                s = src_ch[w, :]
                d = dst_ch[w, :]
                a = plsc.load_gather(as_t, [s]) + plsc.load_gather(ad_t, [d])
                a = jnp.where(a >= 0.0, a, 0.2 * a)
                ex = jnp.exp(a - shift)
                ge = (sid * NV + v) * LN + i16
                ex = jnp.where(ge < EP, ex, 0.0)
                inv = plsc.load_gather(
                    den_l, [lax.shift_right_logical(d, 4),
                            jnp.bitwise_and(d, 15)])
                attn = ex * inv
                attn_ch[w, :] = attn
                m = (d >= lo) & (d < lo + NHALF) & (ge < EP)
                cum = plsc.cumsum(m.astype(jnp.int32))
                pos = off + cum - 1
                plsc.store_scatter(src_c, [pos], s, mask=m)
                plsc.store_scatter(dstl_c, [pos], d - lo, mask=m)
                plsc.store_scatter(attn_c, [pos], attn, mask=m)
                return jnp.minimum(off + jnp.max(cum), CCAP)

            off = lax.fori_loop(0, CHA, _vec, off)

            @pl.when(cid == 0)
            def _():
                pltpu.sync_copy(
                    attn_ch,
                    attn_out.at[pl.ds(sid * NV + t * CHA, CHA), :])

            return off

        off = lax.fori_loop(0, NCHA, _chunk, jnp.int32(0))

        for j in range(GV):
            src_c[pl.ds(off + j * LN, LN)] = zero16i
            dstl_c[pl.ds(off + j * LN, LN)] = zero16i
            attn_c[pl.ds(off + j * LN, LN)] = zero16f
        ngrp = lax.shift_right_logical(off + CG - 1, 5)
        np2 = lax.shift_right_logical(ngrp + 1, 1)

        def _issue(g, buf, sem):
            pltpu.async_copy(
                h_hbm.at[src_c.at[pl.ds(g * CG, CG)]], buf, sem)

        def _process(g, buf, sem):
            base = g * CG
            pltpu.make_async_copy(
                h_hbm.at[src_c.at[pl.ds(base, CG)]], buf, sem).wait()
            for j in range(GV):
                attn_j = attn_c[pl.ds(base + j * LN, LN)]
                for i in range(LN):
                    ai = lax.gather(
                        attn_j, jnp.full((LN, 1), i, jnp.int32),
                        lax.GatherDimensionNumbers(
                            offset_dims=(), collapsed_slice_dims=(0,),
                            start_index_map=(0,)),
                        slice_sizes=(1,),
                        mode=lax.GatherScatterMode.PROMISE_IN_BOUNDS)
                    r = j * LN + i
                    for c in range(8):
                        sl = pl.ds(c * LN, LN)
                        buf[r, sl] = buf[r, sl] * ai
            for j in range(GV):
                dl_j = dstl_c[pl.ds(base + j * LN, LN)]
                pltpu.sync_copy(buf.at[pl.ds(j * LN, LN), :],
                                acc_sh.at[dl_j], add=True)

        @pl.when(ngrp > 0)
        def _():
            _issue(0, rows0, sem0)

        @pl.loop(0, np2)
        def _(k):
            g0 = 2 * k
            g1 = g0 + 1

            @pl.when(g1 < ngrp)
            def _():
                _issue(g1, rows1, sem1)

            _process(g0, rows0, sem0)

            @pl.when(g0 + 2 < ngrp)
            def _():
                _issue(g0 + 2, rows0, sem0)

            @pl.when(g1 < ngrp)
            def _():
                _process(g1, rows1, sem1)

        plsc.subcore_barrier()

        pltpu.sync_copy(acc_sh.at[pl.ds(sid * NROW_BLK, NROW_BLK), :],
                        acc_out.at[cid, pl.ds(sid * NROW_BLK, NROW_BLK), :])

    return sc_layer


_sc_layer = _make_sc_layer()

_BLK = 1000


def _acc_spec():
    return pl.BlockSpec((1, _BLK, D), lambda i: (i // 5, i % 5, 0))


def _tc1_body(x_ref, w_ref, a_ref, h_ref, pair_ref):
    h = jnp.dot(x_ref[...], w_ref[...], preferred_element_type=jnp.float32)
    h_ref[...] = h
    pair_ref[...] = jnp.dot(h, a_ref[...], preferred_element_type=jnp.float32)


def _tc1(x, w, a):
    return pl.pallas_call(
        _tc1_body,
        grid=(N // _BLK,),
        in_specs=[
            pl.BlockSpec((_BLK, D), lambda i: (i, 0)),
            pl.BlockSpec((D, D), lambda i: (0, 0)),
            pl.BlockSpec((D, D), lambda i: (0, 0)),
        ],
        out_specs=[
            pl.BlockSpec((_BLK, D), lambda i: (i, 0)),
            pl.BlockSpec((_BLK, D), lambda i: (i, 0)),
        ],
        out_shape=[
            jax.ShapeDtypeStruct((N, D), jnp.float32),
            jax.ShapeDtypeStruct((N, D), jnp.float32),
        ],
    )(x, w, a)


def _tc2_body(acc_ref, b_ref, w_ref, a_ref, h_ref, pair_ref):
    hin = jax.nn.relu(acc_ref[0] + b_ref[...])
    h = jnp.dot(hin, w_ref[...], preferred_element_type=jnp.float32)
    h_ref[...] = h
    pair_ref[...] = jnp.dot(h, a_ref[...], preferred_element_type=jnp.float32)


def _tc2(acc, b, w, a):
    return pl.pallas_call(
        _tc2_body,
        grid=(N // _BLK,),
        in_specs=[
            _acc_spec(),
            pl.BlockSpec((1, D), lambda i: (0, 0)),
            pl.BlockSpec((D, D), lambda i: (0, 0)),
            pl.BlockSpec((D, D), lambda i: (0, 0)),
        ],
        out_specs=[
            pl.BlockSpec((_BLK, D), lambda i: (i, 0)),
            pl.BlockSpec((_BLK, D), lambda i: (i, 0)),
        ],
        out_shape=[
            jax.ShapeDtypeStruct((N, D), jnp.float32),
            jax.ShapeDtypeStruct((N, D), jnp.float32),
        ],
    )(acc, b, w, a)


def _tc3_body(acc_ref, b_ref, o_ref):
    o_ref[...] = jax.nn.relu(acc_ref[0] + b_ref[...])


def _tc3(acc, b):
    return pl.pallas_call(
        _tc3_body,
        grid=(N // _BLK,),
        in_specs=[
            _acc_spec(),
            pl.BlockSpec((1, D), lambda i: (0, 0)),
        ],
        out_specs=pl.BlockSpec((_BLK, D), lambda i: (i, 0)),
        out_shape=jax.ShapeDtypeStruct((N, D), jnp.float32),
    )(acc, b)


def _proj_mat(a_src, a_dst):
    m = jnp.zeros((D, D), jnp.float32)
    m = m.at[:, 0].set(a_src)
    m = m.at[:, 1].set(a_dst)
    return m


@jax.jit
def kernel(x, edge_index, W1, a_src1, a_dst1, b1, W2, a_src2, a_dst2, b2):
    loop = jnp.arange(N, dtype=jnp.int32)
    pad = jnp.zeros((E_PAD - EP,), jnp.int32)
    src2d = jnp.concatenate([edge_index[0], loop, pad]).reshape(NS * NV, LN)
    dst2d = jnp.concatenate([edge_index[1], loop, pad]).reshape(NS * NV, LN)

    h1, pair1 = _tc1(x, W1, _proj_mat(a_src1, a_dst1))
    acc1, _ = _sc_layer(h1, pair1[:, 0], pair1[:, 1], src2d, dst2d)
    h2, pair2 = _tc2(acc1, b1.reshape(1, D), W2, _proj_mat(a_src2, a_dst2))
    acc2, attn2 = _sc_layer(h2, pair2[:, 0], pair2[:, 1], src2d, dst2d)
    out = _tc3(acc2, b2.reshape(1, D))
    attn = attn2.reshape(-1)[:EP][:, None]
    return out, attn


# speedup vs baseline: 28.0663x; 1.0224x over previous
import functools

import jax
import jax.numpy as jnp
from jax import lax
from jax.experimental import pallas as pl
from jax.experimental.pallas import tpu as pltpu
from jax.experimental.pallas import tpu_sc as plsc

N = 10000
D = 128
E = 320000
EP = E + N
NC = 2
NS = 16
LN = 16
NV = 1312
SA = NV * LN
E_PAD = SA * NS
DEN_ROWS = 640
NHALF = N // 2
ACC_N = 5120
NROW_BLK = ACC_N // NS
CHA = 164
NCHA = NV // CHA
CCAP = 12288
CG = 32
GV = CG // LN

_mesh = plsc.VectorSubcoreMesh(core_axis_name="c", subcore_axis_name="s")

_sc_params = pltpu.CompilerParams(
    needs_layout_passes=False, use_tc_tiling_on_sc=False)


def _make_sc_layer():
    @functools.partial(
        pl.kernel,
        out_type=(
            jax.ShapeDtypeStruct((NC, ACC_N, D), jnp.float32),
            jax.ShapeDtypeStruct((NS * NV, LN), jnp.float32),
        ),
        mesh=_mesh,
        compiler_params=_sc_params,
        scratch_types=[
            pltpu.VMEM((N,), jnp.float32),
            pltpu.VMEM((N,), jnp.float32),
            pltpu.VMEM((DEN_ROWS, LN), jnp.float32),
            pltpu.VMEM((CHA, LN), jnp.int32),
            pltpu.VMEM((CHA, LN), jnp.int32),
            pltpu.VMEM((CHA, LN), jnp.float32),
            pltpu.VMEM((CCAP + CG,), jnp.int32),
            pltpu.VMEM((CCAP + CG,), jnp.int32),
            pltpu.VMEM((CCAP + CG,), jnp.float32),
            pltpu.VMEM((CG, D), jnp.float32),
            pltpu.VMEM((CG, D), jnp.float32),
            pltpu.VMEM((5, 128), jnp.int32),
            pltpu.SemaphoreType.DMA,
            pltpu.SemaphoreType.DMA,
            pltpu.VMEM_SHARED((DEN_ROWS, LN), jnp.float32),
            pltpu.VMEM_SHARED((ACC_N, D), jnp.float32),
        ],
    )
    def sc_layer(h_hbm, as_hbm, ad_hbm, src_hbm, dst_hbm,
                 acc_out, attn_out,
                 as_t, ad_t, den_l, src_ch, dst_ch, attn_ch,
                 src_c, dstl_c, attn_c, rows0, rows1, iota2d,
                 sem0, sem1, den_sh, acc_sh):
        cid = lax.axis_index("c")
        sid = lax.axis_index("s")
        zero16f = jnp.zeros((LN,), jnp.float32)
        zero16i = jnp.zeros((LN,), jnp.int32)
        i16 = jnp.arange(LN, dtype=jnp.int32)

        pltpu.sync_copy(as_hbm, as_t)
        pltpu.sync_copy(ad_hbm, ad_t)

        @pl.loop(0, DEN_ROWS)
        def _(i):
            den_l[i, :] = zero16f

        @pl.loop(0, CG)
        def _(i):
            for c in range(8):
                rows0[i, pl.ds(c * LN, LN)] = zero16f

        for j in range(5):
            for k in range(8):
                iota2d[j, pl.ds(k * LN, LN)] = i16 + (j * 128 + k * LN)

        @pl.when(sid == 0)
        def _():
            pltpu.sync_copy(den_l, den_sh)

        @pl.loop(0, NROW_BLK // CG)
        def _(t):
            pltpu.sync_copy(
                rows0, acc_sh.at[pl.ds(sid * NROW_BLK + t * CG, CG), :])

        plsc.subcore_barrier()

        def _mx(i, m):
            return jnp.maximum(m, as_t[pl.ds(i * LN, LN)])

        def _mx2(i, m):
            return jnp.maximum(m, ad_t[pl.ds(i * LN, LN)])

        neg = jnp.full((LN,), -1e30, jnp.float32)
        ms = jnp.max(lax.fori_loop(0, N // LN, _mx, neg))
        md = jnp.max(lax.fori_loop(0, N // LN, _mx2, neg))
        shift = jnp.maximum(ms + md, 0.0)

        @pl.loop(0, NCHA)
        def _(t):
            pltpu.sync_copy(src_hbm.at[pl.ds(sid * NV + t * CHA, CHA), :],
                            src_ch)
            pltpu.sync_copy(dst_hbm.at[pl.ds(sid * NV + t * CHA, CHA), :],
                            dst_ch)

            @pl.loop(0, CHA)
            def _(w):
                v = t * CHA + w
                s = src_ch[w, :]
                d = dst_ch[w, :]
                a = plsc.load_gather(as_t, [s]) + plsc.load_gather(ad_t, [d])
                a = jnp.where(a >= 0.0, a, 0.2 * a)
                ex = jnp.exp(a - shift)
                ge = (sid * NV + v) * LN + i16
                ex = jnp.where(ge < EP, ex, 0.0)
                plsc.addupdate_scatter(
                    den_l, [lax.shift_right_logical(d, 4),
                            jnp.bitwise_and(d, 15)], ex)

        for j in range(5):
            pltpu.sync_copy(den_l.at[pl.ds(j * 128, 128), :],
                            den_sh.at[iota2d.at[j]], add=True)

        plsc.subcore_barrier()

        pltpu.sync_copy(den_sh, den_l)

        @pl.loop(0, DEN_ROWS)
        def _(i):
            den_l[i, :] = 1.0 / (den_l[i, :] + 1e-16)

        lo = cid * NHALF

        def _chunk(t, off):
            pltpu.sync_copy(src_hbm.at[pl.ds(sid * NV + t * CHA, CHA), :],
                            src_ch)
            pltpu.sync_copy(dst_hbm.at[pl.ds(sid * NV + t * CHA, CHA), :],
                            dst_ch)

            def _vec(w, off):
                v = t * CHA + w
                s = src_ch[w, :]
                d = dst_ch[w, :]
                a = plsc.load_gather(as_t, [s]) + plsc.load_gather(ad_t, [d])
                a = jnp.where(a >= 0.0, a, 0.2 * a)
                ex = jnp.exp(a - shift)
                ge = (sid * NV + v) * LN + i16
                ex = jnp.where(ge < EP, ex, 0.0)
                inv = plsc.load_gather(
                    den_l, [lax.shift_right_logical(d, 4),
                            jnp.bitwise_and(d, 15)])
                attn = ex * inv
                attn_ch[w, :] = attn
                m = (d >= lo) & (d < lo + NHALF) & (ge < EP)
                cum = plsc.cumsum(m.astype(jnp.int32))
                pos = off + cum - 1
                plsc.store_scatter(src_c, [pos], s, mask=m)
                plsc.store_scatter(dstl_c, [pos], d - lo, mask=m)
                plsc.store_scatter(attn_c, [pos], attn, mask=m)
                return jnp.minimum(off + jnp.max(cum), CCAP)

            off = lax.fori_loop(0, CHA, _vec, off)

            @pl.when(cid == 0)
            def _():
                pltpu.sync_copy(
                    attn_ch,
                    attn_out.at[pl.ds(sid * NV + t * CHA, CHA), :])

            return off

        off = lax.fori_loop(0, NCHA, _chunk, jnp.int32(0))

        for j in range(GV):
            src_c[pl.ds(off + j * LN, LN)] = zero16i
            dstl_c[pl.ds(off + j * LN, LN)] = zero16i
            attn_c[pl.ds(off + j * LN, LN)] = zero16f
        ngrp = lax.shift_right_logical(off + CG - 1, 5)
        np2 = lax.shift_right_logical(ngrp + 1, 1)

        def _issue(g, buf, sem):
            pltpu.async_copy(
                h_hbm.at[src_c.at[pl.ds(g * CG, CG)]], buf, sem)

        def _process(g, buf, sem):
            base = g * CG
            pltpu.make_async_copy(
                h_hbm.at[src_c.at[pl.ds(base, CG)]], buf, sem).wait()
            for j in range(GV):
                attn_j = attn_c[pl.ds(base + j * LN, LN)]
                for i in range(LN):
                    ai = lax.gather(
                        attn_j, jnp.full((LN, 1), i, jnp.int32),
                        lax.GatherDimensionNumbers(
                            offset_dims=(), collapsed_slice_dims=(0,),
                            start_index_map=(0,)),
                        slice_sizes=(1,),
                        mode=lax.GatherScatterMode.PROMISE_IN_BOUNDS)
                    r = j * LN + i
                    for c in range(8):
                        sl = pl.ds(c * LN, LN)
                        buf[r, sl] = buf[r, sl] * ai
            pltpu.sync_copy(buf,
                            acc_sh.at[dstl_c.at[pl.ds(base, CG)]], add=True)

        @pl.when(ngrp > 0)
        def _():
            _issue(0, rows0, sem0)

        @pl.loop(0, np2)
        def _(k):
            g0 = 2 * k
            g1 = g0 + 1

            @pl.when(g1 < ngrp)
            def _():
                _issue(g1, rows1, sem1)

            _process(g0, rows0, sem0)

            @pl.when(g0 + 2 < ngrp)
            def _():
                _issue(g0 + 2, rows0, sem0)

            @pl.when(g1 < ngrp)
            def _():
                _process(g1, rows1, sem1)

        plsc.subcore_barrier()

        pltpu.sync_copy(acc_sh.at[pl.ds(sid * NROW_BLK, NROW_BLK), :],
                        acc_out.at[cid, pl.ds(sid * NROW_BLK, NROW_BLK), :])

    return sc_layer


_sc_layer = _make_sc_layer()

_BLK = 1000


def _acc_spec():
    return pl.BlockSpec((1, _BLK, D), lambda i: (i // 5, i % 5, 0))


def _tc1_body(x_ref, w_ref, a_ref, h_ref, pair_ref):
    h = jnp.dot(x_ref[...], w_ref[...], preferred_element_type=jnp.float32)
    h_ref[...] = h
    pair_ref[...] = jnp.dot(h, a_ref[...], preferred_element_type=jnp.float32)


def _tc1(x, w, a):
    return pl.pallas_call(
        _tc1_body,
        grid=(N // _BLK,),
        in_specs=[
            pl.BlockSpec((_BLK, D), lambda i: (i, 0)),
            pl.BlockSpec((D, D), lambda i: (0, 0)),
            pl.BlockSpec((D, D), lambda i: (0, 0)),
        ],
        out_specs=[
            pl.BlockSpec((_BLK, D), lambda i: (i, 0)),
            pl.BlockSpec((_BLK, D), lambda i: (i, 0)),
        ],
        out_shape=[
            jax.ShapeDtypeStruct((N, D), jnp.float32),
            jax.ShapeDtypeStruct((N, D), jnp.float32),
        ],
    )(x, w, a)


def _tc2_body(acc_ref, b_ref, w_ref, a_ref, h_ref, pair_ref):
    hin = jax.nn.relu(acc_ref[0] + b_ref[...])
    h = jnp.dot(hin, w_ref[...], preferred_element_type=jnp.float32)
    h_ref[...] = h
    pair_ref[...] = jnp.dot(h, a_ref[...], preferred_element_type=jnp.float32)


def _tc2(acc, b, w, a):
    return pl.pallas_call(
        _tc2_body,
        grid=(N // _BLK,),
        in_specs=[
            _acc_spec(),
            pl.BlockSpec((1, D), lambda i: (0, 0)),
            pl.BlockSpec((D, D), lambda i: (0, 0)),
            pl.BlockSpec((D, D), lambda i: (0, 0)),
        ],
        out_specs=[
            pl.BlockSpec((_BLK, D), lambda i: (i, 0)),
            pl.BlockSpec((_BLK, D), lambda i: (i, 0)),
        ],
        out_shape=[
            jax.ShapeDtypeStruct((N, D), jnp.float32),
            jax.ShapeDtypeStruct((N, D), jnp.float32),
        ],
    )(acc, b, w, a)


def _tc3_body(acc_ref, b_ref, o_ref):
    o_ref[...] = jax.nn.relu(acc_ref[0] + b_ref[...])


def _tc3(acc, b):
    return pl.pallas_call(
        _tc3_body,
        grid=(N // _BLK,),
        in_specs=[
            _acc_spec(),
            pl.BlockSpec((1, D), lambda i: (0, 0)),
        ],
        out_specs=pl.BlockSpec((_BLK, D), lambda i: (i, 0)),
        out_shape=jax.ShapeDtypeStruct((N, D), jnp.float32),
    )(acc, b)


def _proj_mat(a_src, a_dst):
    m = jnp.zeros((D, D), jnp.float32)
    m = m.at[:, 0].set(a_src)
    m = m.at[:, 1].set(a_dst)
    return m


@jax.jit
def kernel(x, edge_index, W1, a_src1, a_dst1, b1, W2, a_src2, a_dst2, b2):
    loop = jnp.arange(N, dtype=jnp.int32)
    pad = jnp.zeros((E_PAD - EP,), jnp.int32)
    src2d = jnp.concatenate([edge_index[0], loop, pad]).reshape(NS * NV, LN)
    dst2d = jnp.concatenate([edge_index[1], loop, pad]).reshape(NS * NV, LN)

    h1, pair1 = _tc1(x, W1, _proj_mat(a_src1, a_dst1))
    acc1, _ = _sc_layer(h1, pair1[:, 0], pair1[:, 1], src2d, dst2d)
    h2, pair2 = _tc2(acc1, b1.reshape(1, D), W2, _proj_mat(a_src2, a_dst2))
    acc2, attn2 = _sc_layer(h2, pair2[:, 0], pair2[:, 1], src2d, dst2d)
    out = _tc3(acc2, b2.reshape(1, D))
    attn = attn2.reshape(-1)[:EP][:, None]
    return out, attn
